# baseline (speedup 1.0000x reference)
import functools

import jax
import jax.numpy as jnp
from jax import lax
from jax.experimental import pallas as pl
from jax.experimental.pallas import tpu as pltpu
from jax.experimental.pallas import tpu_sc as plsc

D_MODEL = 64
N_S = 50
S_PAD = 56
NUM_CORES = 2
NUM_SUBCORES = 16
NUM_WORKERS = NUM_CORES * NUM_SUBCORES
B_CHUNK = 8


@functools.partial(jax.jit, static_argnums=(2,))
def _gather_rows(idx, table, n_b):
    rows_per_chunk = B_CHUNK * N_S
    b_per_w = n_b // NUM_WORKERS
    n_chunks = b_per_w // B_CHUNK
    assert n_chunks % 2 == 0
    idx2 = idx.reshape(NUM_WORKERS, n_chunks, rows_per_chunk)
    mesh = plsc.VectorSubcoreMesh(core_axis_name="c", subcore_axis_name="s")

    @functools.partial(
        pl.kernel,
        mesh=mesh,
        out_type=jax.ShapeDtypeStruct((n_b, S_PAD, 128), jnp.float32),
        scratch_types=[
            pltpu.VMEM((n_chunks, rows_per_chunk), jnp.int32),
            pltpu.VMEM((2, rows_per_chunk, D_MODEL), jnp.float32),
            pltpu.SemaphoreType.DMA,
            pltpu.SemaphoreType.DMA,
        ],
        compiler_params=pltpu.CompilerParams(use_tc_tiling_on_sc=False),
    )
    def k(idx_hbm, table_hbm, out_hbm, idx_v, rows_v, g_sem, o_sem):
        wid = lax.axis_index("s") * NUM_CORES + lax.axis_index("c")
        base_b = wid * b_per_w
        pltpu.sync_copy(idx_hbm.at[wid], idx_v)
        pltpu.async_copy(table_hbm.at[idx_v.at[0]], rows_v.at[0], g_sem)

        def out_writes(i, s):
            for j in range(B_CHUNK):
                pltpu.async_copy(
                    rows_v.at[s, pl.ds(j * N_S, N_S)],
                    out_hbm.at[
                        base_b + i * B_CHUNK + j, pl.ds(0, N_S), pl.ds(0, D_MODEL)
                    ],
                    o_sem,
                )

        def wait_out_writes(s):
            pltpu.make_async_copy(
                rows_v.at[s],
                table_hbm.at[pl.ds(0, B_CHUNK * N_S)],
                o_sem,
            ).wait()

        def step(i, s, s_next):
            @pl.when(i >= 1)
            def _():
                wait_out_writes(s_next)

            @pl.when(i + 1 < n_chunks)
            def _():
                pltpu.async_copy(
                    table_hbm.at[idx_v.at[i + 1]], rows_v.at[s_next], g_sem
                )

            pltpu.make_async_copy(
                table_hbm.at[idx_v.at[i]], rows_v.at[s], g_sem
            ).wait()
            out_writes(i, s)

        def body(p, carry):
            step(2 * p, 0, 1)
            step(2 * p + 1, 1, 0)
            return carry

        lax.fori_loop(0, n_chunks // 2, body, 0)
        wait_out_writes(1)

    return k(idx2, table)


DETILE_BLK = 2048


@jax.jit
def _detile_table(table_t):
    _, v = table_t.shape
    grid = (v + DETILE_BLK - 1) // DETILE_BLK

    def body(in_ref, out_ref):
        x = in_ref[...]
        row = jax.lax.broadcasted_iota(jnp.int32, (64, 64), 0)
        col = jax.lax.broadcasted_iota(jnp.int32, (64, 64), 1)
        eye = (row == col).astype(jnp.float32)
        y = jax.lax.dot_general(
            x, eye, (((0,), (0,)), ((), ())),
            preferred_element_type=jnp.float32,
        )
        z = y.reshape(DETILE_BLK // 2, 2, 64)
        out_ref[...] = jnp.concatenate([z[:, 0, :], z[:, 1, :]], axis=1)

    return pl.pallas_call(
        body,
        grid=(grid,),
        in_specs=[pl.BlockSpec((64, DETILE_BLK), lambda i: (0, i))],
        out_specs=pl.BlockSpec((DETILE_BLK // 2, 128), lambda i: (i, 0)),
        out_shape=jax.ShapeDtypeStruct((v // 2, 128), jnp.float32),
    )(table_t)


def kernel(x, table):
    b, s = x.shape
    v, d = table.shape
    idx = x.reshape(b * s).astype(jnp.int32)
    table_lin = _detile_table(table.T).reshape(v, d)
    out_padded = _gather_rows(idx, table_lin, b)
    return out_padded[:, :N_S, :D_MODEL]


# speedup vs baseline: 1.2221x; 1.2221x over previous
import functools

import jax
import jax.numpy as jnp
from jax import lax
from jax.experimental import pallas as pl
from jax.experimental.pallas import tpu as pltpu
from jax.experimental.pallas import tpu_sc as plsc

D_MODEL = 64
N_S = 50
S_PAD = 56
NUM_CORES = 2
NUM_SUBCORES = 16
NUM_WORKERS = NUM_CORES * NUM_SUBCORES
B_CHUNK = 8


@functools.partial(jax.jit, static_argnums=(2,))
def _gather_rows(idx, table, n_b):
    rows_per_chunk = B_CHUNK * N_S
    b_per_w = n_b // NUM_WORKERS
    n_chunks = b_per_w // B_CHUNK
    assert n_chunks % 2 == 0
    idx2 = idx.reshape(NUM_WORKERS, n_chunks, rows_per_chunk)
    mesh = plsc.VectorSubcoreMesh(core_axis_name="c", subcore_axis_name="s")

    @functools.partial(
        pl.kernel,
        mesh=mesh,
        out_type=jax.ShapeDtypeStruct((n_b, S_PAD, 128), jnp.float32),
        scratch_types=[
            pltpu.VMEM((n_chunks, rows_per_chunk), jnp.int32),
            pltpu.VMEM((2, rows_per_chunk, D_MODEL), jnp.float32),
            pltpu.SemaphoreType.DMA,
            pltpu.SemaphoreType.DMA,
        ],
        compiler_params=pltpu.CompilerParams(use_tc_tiling_on_sc=False),
    )
    def k(idx_hbm, table_hbm, out_hbm, idx_v, rows_v, g_sem, o_sem):
        wid = lax.axis_index("s") * NUM_CORES + lax.axis_index("c")
        base_b = wid * b_per_w
        pltpu.sync_copy(idx_hbm.at[wid], idx_v)
        pltpu.async_copy(table_hbm.at[idx_v.at[0]], rows_v.at[0], g_sem)

        def out_writes(i, s):
            for j in range(B_CHUNK):
                pltpu.async_copy(
                    rows_v.at[s, pl.ds(j * N_S, N_S)],
                    out_hbm.at[
                        base_b + i * B_CHUNK + j, pl.ds(0, N_S), pl.ds(0, D_MODEL)
                    ],
                    o_sem,
                )

        def wait_out_writes(s):
            pltpu.make_async_copy(
                rows_v.at[s],
                table_hbm.at[pl.ds(0, B_CHUNK * N_S)],
                o_sem,
            ).wait()

        def step(i, s, s_next):
            @pl.when(i >= 1)
            def _():
                wait_out_writes(s_next)

            @pl.when(i + 1 < n_chunks)
            def _():
                pltpu.async_copy(
                    table_hbm.at[idx_v.at[i + 1]], rows_v.at[s_next], g_sem
                )

            pltpu.make_async_copy(
                table_hbm.at[idx_v.at[i]], rows_v.at[s], g_sem
            ).wait()
            out_writes(i, s)

        def body(p, carry):
            step(2 * p, 0, 1)
            step(2 * p + 1, 1, 0)
            return carry

        lax.fori_loop(0, n_chunks // 2, body, 0)
        wait_out_writes(1)

    return k(idx2, table)


DETILE_BLK = 8192


@jax.jit
def _detile_table(table_t):
    _, v = table_t.shape
    grid = (v + DETILE_BLK - 1) // DETILE_BLK

    def body(in_ref, out_ref):
        x = in_ref[...]
        y = jnp.transpose(x)
        z = y.reshape(DETILE_BLK // 2, 2, 64)
        out_ref[...] = jnp.concatenate([z[:, 0, :], z[:, 1, :]], axis=1)

    return pl.pallas_call(
        body,
        grid=(grid,),
        in_specs=[pl.BlockSpec((64, DETILE_BLK), lambda i: (0, i))],
        out_specs=pl.BlockSpec((DETILE_BLK // 2, 128), lambda i: (i, 0)),
        out_shape=jax.ShapeDtypeStruct((v // 2, 128), jnp.float32),
    )(table_t)


def kernel(x, table):
    b, s = x.shape
    v, d = table.shape
    idx = x.reshape(b * s).astype(jnp.int32)
    table_lin = _detile_table(table.T).reshape(v, d)
    out_padded = _gather_rows(idx, table_lin, b)
    return out_padded[:, :N_S, :D_MODEL]


# speedup vs baseline: 1.2267x; 1.0037x over previous
import functools

import jax
import jax.numpy as jnp
from jax import lax
from jax.experimental import pallas as pl
from jax.experimental.pallas import tpu as pltpu
from jax.experimental.pallas import tpu_sc as plsc

D_MODEL = 64
N_S = 50
S_PAD = 56
NUM_CORES = 2
NUM_SUBCORES = 16
NUM_WORKERS = NUM_CORES * NUM_SUBCORES
B_CHUNK = 8


@functools.partial(jax.jit, static_argnums=(2,))
def _gather_rows(idx, table, n_b):
    rows_per_chunk = B_CHUNK * N_S
    b_per_w = n_b // NUM_WORKERS
    n_chunks = b_per_w // B_CHUNK
    assert n_chunks % 2 == 0
    idx2 = idx.reshape(NUM_WORKERS, n_chunks, rows_per_chunk)
    mesh = plsc.VectorSubcoreMesh(core_axis_name="c", subcore_axis_name="s")

    @functools.partial(
        pl.kernel,
        mesh=mesh,
        out_type=jax.ShapeDtypeStruct((n_b, S_PAD, 128), jnp.float32),
        scratch_types=[
            pltpu.VMEM((n_chunks, rows_per_chunk), jnp.int32),
            pltpu.VMEM((2, rows_per_chunk, D_MODEL), jnp.float32),
            pltpu.SemaphoreType.DMA,
            pltpu.SemaphoreType.DMA,
        ],
        compiler_params=pltpu.CompilerParams(use_tc_tiling_on_sc=False),
    )
    def k(idx_hbm, table_hbm, out_hbm, idx_v, rows_v, g_sem, o_sem):
        wid = lax.axis_index("s") * NUM_CORES + lax.axis_index("c")
        base_b = wid * b_per_w
        pltpu.sync_copy(idx_hbm.at[wid], idx_v)
        pltpu.async_copy(table_hbm.at[idx_v.at[0]], rows_v.at[0], g_sem)

        def out_writes(i, s):
            for j in range(B_CHUNK):
                pltpu.async_copy(
                    rows_v.at[s, pl.ds(j * N_S, N_S)],
                    out_hbm.at[
                        base_b + i * B_CHUNK + j, pl.ds(0, N_S), pl.ds(0, D_MODEL)
                    ],
                    o_sem,
                )

        def wait_out_writes(s):
            pltpu.make_async_copy(
                rows_v.at[s],
                table_hbm.at[pl.ds(0, B_CHUNK * N_S)],
                o_sem,
            ).wait()

        def step(i, s, s_next):
            @pl.when(i >= 1)
            def _():
                wait_out_writes(s_next)

            @pl.when(i + 1 < n_chunks)
            def _():
                pltpu.async_copy(
                    table_hbm.at[idx_v.at[i + 1]], rows_v.at[s_next], g_sem
                )

            pltpu.make_async_copy(
                table_hbm.at[idx_v.at[i]], rows_v.at[s], g_sem
            ).wait()
            out_writes(i, s)

        def body(p, carry):
            step(2 * p, 0, 1)
            step(2 * p + 1, 1, 0)
            return carry

        lax.fori_loop(0, n_chunks // 2, body, 0)
        wait_out_writes(1)

    return k(idx2, table)


DETILE_BLK = 16384


@jax.jit
def _detile_table(table_t):
    _, v = table_t.shape
    grid = (v + DETILE_BLK - 1) // DETILE_BLK

    def body(in_ref, out_ref):
        x = in_ref[...]
        y = jnp.transpose(x)
        z = y.reshape(DETILE_BLK // 2, 2, 64)
        out_ref[...] = jnp.concatenate([z[:, 0, :], z[:, 1, :]], axis=1)

    return pl.pallas_call(
        body,
        grid=(grid,),
        in_specs=[pl.BlockSpec((64, DETILE_BLK), lambda i: (0, i))],
        out_specs=pl.BlockSpec((DETILE_BLK // 2, 128), lambda i: (i, 0)),
        out_shape=jax.ShapeDtypeStruct((v // 2, 128), jnp.float32),
    )(table_t)


def kernel(x, table):
    b, s = x.shape
    v, d = table.shape
    idx = x.reshape(b * s).astype(jnp.int32)
    table_lin = _detile_table(table.T).reshape(v, d)
    out_padded = _gather_rows(idx, table_lin, b)
    return out_padded[:, :N_S, :D_MODEL]
